# Initial kernel scaffold; baseline (speedup 1.0000x reference)
import jax
import jax.numpy as jnp
from jax.experimental import pallas as pl


def kernel(x_a, edge_index_a, batch_a, x_b, edge_index_b, batch_b, W_gcn, b_gcn, W_gat, a_src, a_dst, b_gat, W_fcg1, b_fcg1, W_fcg2, b_fcg2, W_fc1, b_fc1, W_fc2, b_fc2, W_out, b_out):
    raise NotImplementedError("write your pallas kernel here")


# speedup vs baseline: 1.0069x; 1.0069x over previous
import functools

import jax
import jax.numpy as jnp
from jax.experimental import pallas as pl

_N = 10000
_HEADS = 10
_HDIM = 32
_NGRAPH = 512


def _mm_kernel(x_ref, w_ref, b_ref, o_ref, *, act):
    acc = jnp.dot(x_ref[...], w_ref[...], preferred_element_type=jnp.float32)
    acc = acc + b_ref[...]
    if act:
        acc = jnp.maximum(acc, 0.0)
    o_ref[...] = acc


def _mm(x, w, b, act=False, bm=None):
    m, k = x.shape
    n = w.shape[1]
    if bm is None:
        bm = m
    b2 = b.reshape(1, n)
    return pl.pallas_call(
        functools.partial(_mm_kernel, act=act),
        grid=(pl.cdiv(m, bm),),
        in_specs=[
            pl.BlockSpec((bm, k), lambda i: (i, 0)),
            pl.BlockSpec((k, n), lambda i: (0, 0)),
            pl.BlockSpec((1, n), lambda i: (0, 0)),
        ],
        out_specs=pl.BlockSpec((bm, n), lambda i: (i, 0)),
        out_shape=jax.ShapeDtypeStruct((m, n), jnp.float32),
    )(x, w, b2)


def _self_loops(ei):
    sl = jnp.arange(_N, dtype=ei.dtype)
    return jnp.concatenate([ei[0], sl]), jnp.concatenate([ei[1], sl])


def _branch(x, ei, batch, W_gcn, b_gcn, W_gat, a_src, a_dst, b_gat,
            W_fcg1, b_fcg1, W_fcg2, b_fcg2):
    src, dst = _self_loops(ei)
    ones = jnp.ones_like(dst, dtype=jnp.float32)

    hw = _mm(x, W_gcn, jnp.zeros_like(b_gcn), act=False, bm=1000)
    deg = jax.ops.segment_sum(ones, dst, num_segments=_N)
    dinv = 1.0 / jnp.sqrt(jnp.maximum(deg, 1.0))
    norm = dinv[src] * dinv[dst]
    msg = hw[src] * norm[:, None]
    h = jnp.maximum(jax.ops.segment_sum(msg, dst, num_segments=_N) + b_gcn, 0.0)

    hg = _mm(h, W_gat, jnp.zeros_like(b_gat), act=False, bm=1000)
    hg3 = hg.reshape(_N, _HEADS, _HDIM)
    e_s = jnp.sum(hg3 * a_src[None, :, :], axis=-1)
    e_d = jnp.sum(hg3 * a_dst[None, :, :], axis=-1)
    e = jax.nn.leaky_relu(e_s[src] + e_d[dst], negative_slope=0.2)
    emax = jax.ops.segment_max(e, dst, num_segments=_N)
    emax = jnp.where(jnp.isfinite(emax), emax, 0.0)
    ex = jnp.exp(e - emax[dst])
    denom = jax.ops.segment_sum(ex, dst, num_segments=_N)
    alpha = ex / jnp.maximum(denom[dst], 1e-16)
    gmsg = hg3[src] * alpha[:, :, None]
    agg = jax.ops.segment_sum(gmsg, dst, num_segments=_N).reshape(_N, _HEADS * _HDIM)
    h2 = jnp.maximum(agg + b_gat, 0.0)

    bs = jax.ops.segment_sum(h2, batch, num_segments=_NGRAPH)
    cnt = jax.ops.segment_sum(jnp.ones((_N,), jnp.float32), batch, num_segments=_NGRAPH)
    mean = bs / jnp.maximum(cnt, 1.0)[:, None]
    mx = jax.ops.segment_max(h2, batch, num_segments=_NGRAPH)
    mx = jnp.where(jnp.isfinite(mx), mx, 0.0)
    g = jnp.concatenate([mx, mean], axis=1)

    g = _mm(g, W_fcg1, b_fcg1, act=True)
    g = _mm(g, W_fcg2, b_fcg2, act=False)
    return g


def kernel(x_a, edge_index_a, batch_a, x_b, edge_index_b, batch_b, W_gcn, b_gcn,
           W_gat, a_src, a_dst, b_gat, W_fcg1, b_fcg1, W_fcg2, b_fcg2, W_fc1,
           b_fc1, W_fc2, b_fc2, W_out, b_out):
    xa = _branch(x_a, edge_index_a, batch_a, W_gcn, b_gcn, W_gat, a_src, a_dst,
                 b_gat, W_fcg1, b_fcg1, W_fcg2, b_fcg2)
    xb = _branch(x_b, edge_index_b, batch_b, W_gcn, b_gcn, W_gat, a_src, a_dst,
                 b_gat, W_fcg1, b_fcg1, W_fcg2, b_fcg2)
    xc = jnp.concatenate([xa, xb], axis=1)
    xc = _mm(xc, W_fc1, b_fc1, act=True)
    xc = _mm(xc, W_fc2, b_fc2, act=True)
    return _mm(xc, W_out, b_out, act=False)


# speedup vs baseline: 1.0769x; 1.0695x over previous
import functools

import jax
import jax.numpy as jnp
from jax.experimental import pallas as pl

_N = 10000
_HEADS = 10
_HDIM = 32
_NGRAPH = 512


def _mm_kernel(x_ref, w_ref, b_ref, o_ref, *, act):
    acc = jnp.dot(x_ref[...], w_ref[...], preferred_element_type=jnp.float32)
    acc = acc + b_ref[...]
    if act:
        acc = jnp.maximum(acc, 0.0)
    o_ref[...] = acc


def _mm(x, w, b, act=False, bm=None):
    m, k = x.shape
    n = w.shape[1]
    if bm is None:
        bm = m
    b2 = b.reshape(1, n)
    return pl.pallas_call(
        functools.partial(_mm_kernel, act=act),
        grid=(pl.cdiv(m, bm),),
        in_specs=[
            pl.BlockSpec((bm, k), lambda i: (i, 0)),
            pl.BlockSpec((k, n), lambda i: (0, 0)),
            pl.BlockSpec((1, n), lambda i: (0, 0)),
        ],
        out_specs=pl.BlockSpec((bm, n), lambda i: (i, 0)),
        out_shape=jax.ShapeDtypeStruct((m, n), jnp.float32),
    )(x, w, b2)


def _self_loops(ei):
    sl = jnp.arange(_N, dtype=ei.dtype)
    return jnp.concatenate([ei[0], sl]), jnp.concatenate([ei[1], sl])


def _branch(x, ei, batch, W_gcn, b_gcn, W_gat, a_src, a_dst, b_gat,
            W_fcg1, b_fcg1, W_fcg2, b_fcg2):
    src, dst = _self_loops(ei)
    ones = jnp.ones_like(dst, dtype=jnp.float32)

    hw = _mm(x, W_gcn, jnp.zeros_like(b_gcn), act=False, bm=1000)
    deg = jax.ops.segment_sum(ones, dst, num_segments=_N)
    dinv = 1.0 / jnp.sqrt(jnp.maximum(deg, 1.0))
    agg0 = jax.ops.segment_sum((hw * dinv[:, None])[src], dst, num_segments=_N)
    h = jnp.maximum(agg0 * dinv[:, None] + b_gcn, 0.0)

    hg = _mm(h, W_gat, jnp.zeros_like(b_gat), act=False, bm=1000)
    hg3 = hg.reshape(_N, _HEADS, _HDIM)
    e_s = jnp.sum(hg3 * a_src[None, :, :], axis=-1)
    e_d = jnp.sum(hg3 * a_dst[None, :, :], axis=-1)
    e = jax.nn.leaky_relu(e_s[src] + e_d[dst], negative_slope=0.2)
    emax = jax.ops.segment_max(e, dst, num_segments=_N)
    emax = jnp.where(jnp.isfinite(emax), emax, 0.0)
    ex = jnp.exp(e - emax[dst])
    denom = jax.ops.segment_sum(ex, dst, num_segments=_N)
    alpha = ex / jnp.maximum(denom[dst], 1e-16)
    gmsg = hg3[src] * alpha[:, :, None]
    agg = jax.ops.segment_sum(gmsg, dst, num_segments=_N).reshape(_N, _HEADS * _HDIM)
    h2 = jnp.maximum(agg + b_gat, 0.0)

    bs = jax.ops.segment_sum(h2, batch, num_segments=_NGRAPH)
    cnt = jax.ops.segment_sum(jnp.ones((_N,), jnp.float32), batch, num_segments=_NGRAPH)
    mean = bs / jnp.maximum(cnt, 1.0)[:, None]
    mx = jax.ops.segment_max(h2, batch, num_segments=_NGRAPH)
    mx = jnp.where(jnp.isfinite(mx), mx, 0.0)
    g = jnp.concatenate([mx, mean], axis=1)

    g = _mm(g, W_fcg1, b_fcg1, act=True)
    g = _mm(g, W_fcg2, b_fcg2, act=False)
    return g


def kernel(x_a, edge_index_a, batch_a, x_b, edge_index_b, batch_b, W_gcn, b_gcn,
           W_gat, a_src, a_dst, b_gat, W_fcg1, b_fcg1, W_fcg2, b_fcg2, W_fc1,
           b_fc1, W_fc2, b_fc2, W_out, b_out):
    xa = _branch(x_a, edge_index_a, batch_a, W_gcn, b_gcn, W_gat, a_src, a_dst,
                 b_gat, W_fcg1, b_fcg1, W_fcg2, b_fcg2)
    xb = _branch(x_b, edge_index_b, batch_b, W_gcn, b_gcn, W_gat, a_src, a_dst,
                 b_gat, W_fcg1, b_fcg1, W_fcg2, b_fcg2)
    xc = jnp.concatenate([xa, xb], axis=1)
    xc = _mm(xc, W_fc1, b_fc1, act=True)
    xc = _mm(xc, W_fc2, b_fc2, act=True)
    return _mm(xc, W_out, b_out, act=False)
